# baseline (speedup 1.0000x reference)
import functools

import jax
import jax.numpy as jnp
from jax import lax
from jax.experimental import pallas as pl
from jax.experimental.pallas import tpu as pltpu
from jax.experimental.pallas import tpu_sc as plsc

_NUM_EMBEDDINGS = 32000
_DIM = 128
_BATCH = 4
_SEQ = 8192
_N = _BATCH * _SEQ

_info = plsc.get_sparse_core_info()
_NC = _info.num_cores
_NS = _info.num_subcores
_NW = _NC * _NS
_BPW = _N // _NW
_CHUNK = 128
_NCHUNK = _BPW // _CHUNK

_mesh = plsc.VectorSubcoreMesh(core_axis_name="c", subcore_axis_name="s")


_NBUF = 7
_WPS = _SEQ // _BPW
_HALF = _BPW // 2


@functools.partial(
    pl.kernel,
    mesh=_mesh,
    out_type=jax.ShapeDtypeStruct((_N, _DIM), jnp.float32),
    scratch_types=[
        pltpu.VMEM((_BPW,), jnp.int32),
        *[pltpu.VMEM((_CHUNK, _DIM), jnp.float32) for _ in range(_NBUF)],
        *[pltpu.SemaphoreType.DMA for _ in range(2 * _NBUF + 1)],
    ],
)
def _embed_sc(idx_hbm, table_hbm, out_hbm, idx_v, *bufs_and_sems):
    rows = bufs_and_sems[:_NBUF]
    gsem = bufs_and_sems[_NBUF:2 * _NBUF]
    ssem = bufs_and_sems[2 * _NBUF:3 * _NBUF]
    isem = bufs_and_sems[3 * _NBUF]

    wid = lax.axis_index("s") * _NC + lax.axis_index("c")
    base = wid * _BPW
    col = (wid % _WPS) * _BPW
    row = wid // _WPS

    ih2 = pltpu.async_copy(
        idx_hbm.at[row, pl.ds(col + _HALF, _HALF)],
        idx_v.at[pl.ds(_HALF, _HALF)], isem)
    pltpu.sync_copy(idx_hbm.at[row, pl.ds(col, _HALF)],
                    idx_v.at[pl.ds(0, _HALF)])

    def gather(j):
        b = j % _NBUF
        return pltpu.async_copy(
            table_hbm.at[idx_v.at[pl.ds(j * _CHUNK, _CHUNK)]],
            rows[b], gsem[b])

    def store(j):
        b = j % _NBUF
        return pltpu.async_copy(
            rows[b], out_hbm.at[pl.ds(base + j * _CHUNK, _CHUNK)], ssem[b])

    half_chunk = _HALF // _CHUNK
    gathers = []
    for j in range(_NBUF):
        if j == half_chunk:
            ih2.wait()
        gathers.append(gather(j))
    stores = [None] * _NCHUNK
    for j in range(_NCHUNK):
        gathers[j].wait()
        stores[j] = store(j)
        nxt = j + _NBUF
        if nxt < _NCHUNK:
            stores[j].wait()
            gathers.append(gather(nxt))
    for j in range(_NCHUNK - _NBUF, _NCHUNK):
        stores[j].wait()


def kernel(input_, weight):
    out = _embed_sc(input_, weight)
    return out.reshape(_BATCH, _SEQ, _DIM)


# speedup vs baseline: 1.0189x; 1.0189x over previous
import functools

import jax
import jax.numpy as jnp
from jax import lax
from jax.experimental import pallas as pl
from jax.experimental.pallas import tpu as pltpu
from jax.experimental.pallas import tpu_sc as plsc

_NUM_EMBEDDINGS = 32000
_DIM = 128
_BATCH = 4
_SEQ = 8192
_N = _BATCH * _SEQ

_info = plsc.get_sparse_core_info()
_NC = _info.num_cores
_NS = _info.num_subcores
_NW = _NC * _NS
_BPW = _N // _NW
_CHUNK = 128
_NCHUNK = _BPW // _CHUNK

_mesh = plsc.VectorSubcoreMesh(core_axis_name="c", subcore_axis_name="s")


_NBUF = 6
_WPS = _SEQ // _BPW


@functools.partial(
    pl.kernel,
    mesh=_mesh,
    out_type=jax.ShapeDtypeStruct((_N, _DIM), jnp.float32),
    scratch_types=[
        pltpu.VMEM((_BPW,), jnp.int32),
        *[pltpu.VMEM((_CHUNK, _DIM), jnp.float32) for _ in range(_NBUF)],
        *[pltpu.SemaphoreType.DMA for _ in range(2 * _NBUF)],
    ],
)
def _embed_sc(idx_hbm, table_hbm, out_hbm, idx_v, *bufs_and_sems):
    rows = bufs_and_sems[:_NBUF]
    gsem = bufs_and_sems[_NBUF:2 * _NBUF]
    ssem = bufs_and_sems[2 * _NBUF:]

    wid = lax.axis_index("s") * _NC + lax.axis_index("c")
    base = wid * _BPW

    pltpu.sync_copy(
        idx_hbm.at[wid // _WPS, pl.ds((wid % _WPS) * _BPW, _BPW)], idx_v)

    def gather(j):
        b = j % _NBUF
        return pltpu.async_copy(
            table_hbm.at[idx_v.at[pl.ds(j * _CHUNK, _CHUNK)]],
            rows[b], gsem[b])

    def store(j):
        b = j % _NBUF
        return pltpu.async_copy(
            rows[b], out_hbm.at[pl.ds(base + j * _CHUNK, _CHUNK)], ssem[b])

    gathers = [gather(j) for j in range(_NBUF)]
    stores = [None] * _NCHUNK
    for j in range(_NCHUNK):
        gathers[j].wait()
        stores[j] = store(j)
        nxt = j + _NBUF
        if nxt < _NCHUNK:
            stores[j].wait()
            gathers.append(gather(nxt))
    for j in range(_NCHUNK - _NBUF, _NCHUNK):
        stores[j].wait()


def kernel(input_, weight):
    out = _embed_sc(input_, weight)
    return out.reshape(_BATCH, _SEQ, _DIM)
